# baseline (speedup 1.0000x reference)
import functools

import jax
import jax.numpy as jnp
from jax import lax
from jax.experimental import pallas as pl
from jax.experimental.pallas import tpu as pltpu
from jax.experimental.pallas import tpu_sc as plsc

N = 10000
E = 320000
G = 64
D = 128
DO = 64

NC = 2
NS = 16
K = 128
T = 80
EP = NC * NS * T * K
ECH = EP // K
NP = 10240
RPT = NP // NS

_mesh = plsc.VectorSubcoreMesh(core_axis_name="c", subcore_axis_name="s")


@functools.partial(
    pl.kernel,
    mesh=_mesh,
    out_type=jax.ShapeDtypeStruct((NC, NP, 16), jnp.float32),
    scratch_types=[
        pltpu.VMEM_SHARED((NP, 16), jnp.float32),
        pltpu.VMEM((K, 16), jnp.float32),
        pltpu.VMEM((K,), jnp.int32),
    ],
)
def _sc_degree(dst_hbm, out_hbm, acc_sh, ones_v, didx_v):
    c = lax.axis_index("c")
    s = lax.axis_index("s")
    base = (c * NS + s) * T * K

    @pl.loop(0, K)
    def _(r):
        ones_v[r, :] = jnp.zeros((16,), jnp.float32)

    @pl.loop(0, RPT // K)
    def _(j):
        pltpu.sync_copy(ones_v, acc_sh.at[pl.ds(s * RPT + j * K, K)])

    @pl.loop(0, K)
    def _(r):
        ones_v[r, :] = jnp.ones((16,), jnp.float32)

    plsc.subcore_barrier()

    @pl.loop(0, T)
    def _(i):
        pltpu.sync_copy(dst_hbm.at[pl.ds(base + i * K, K)], didx_v)
        pltpu.sync_copy(ones_v, acc_sh.at[didx_v], add=True)

    plsc.subcore_barrier()
    pltpu.sync_copy(acc_sh.at[pl.ds(s * RPT, RPT)],
                    out_hbm.at[c].at[pl.ds(s * RPT, RPT)])


@functools.partial(
    pl.kernel,
    mesh=_mesh,
    out_type=jax.ShapeDtypeStruct((NC, NP, D), jnp.float32),
    scratch_types=[
        pltpu.VMEM_SHARED((NP, D), jnp.float32),
        pltpu.VMEM((K, D), jnp.float32),
        pltpu.VMEM((K, D), jnp.float32),
        pltpu.VMEM((K,), jnp.int32),
        pltpu.VMEM((K,), jnp.int32),
        pltpu.VMEM((K,), jnp.int32),
        pltpu.VMEM((K,), jnp.int32),
        pltpu.VMEM((K,), jnp.int32),
        pltpu.VMEM((K,), jnp.int32),
        pltpu.VMEM((K,), jnp.int32),
        pltpu.VMEM((K,), jnp.int32),
        pltpu.SemaphoreType.DMA,
        pltpu.SemaphoreType.DMA,
        pltpu.SemaphoreType.DMA,
        pltpu.SemaphoreType.DMA,
        pltpu.SemaphoreType.DMA,
        pltpu.SemaphoreType.DMA,
    ],
)
def _sc_aggregate(hs_hbm, src_hbm, dst_hbm, out_hbm,
                  acc_sh, rows0, rows1,
                  sx0, sx1, sx2, sx3, dx0, dx1, dx2, dx3,
                  gs0, gs1, is0, is1, is2, is3):
    c = lax.axis_index("c")
    s = lax.axis_index("s")
    base = (c * NS + s) * T * K

    sxs = (sx0, sx1, sx2, sx3)
    dxs = (dx0, dx1, dx2, dx3)
    iss = (is0, is1, is2, is3)
    rbs = (rows0, rows1)
    gss = (gs0, gs1)

    def idx_load(j, slot):
        pltpu.async_copy(src_hbm.at[pl.ds(base + j * K, K)], sxs[slot],
                         iss[slot])
        pltpu.async_copy(dst_hbm.at[pl.ds(base + j * K, K)], dxs[slot],
                         iss[slot])

    def idx_wait(j, slot):
        pltpu.make_async_copy(src_hbm.at[pl.ds(base + j * K, K)], sxs[slot],
                              iss[slot]).wait()
        pltpu.make_async_copy(dst_hbm.at[pl.ds(base + j * K, K)], dxs[slot],
                              iss[slot]).wait()

    @pl.loop(0, K)
    def _(r):
        @pl.loop(0, D // 16)
        def _(j):
            rows0[r, pl.ds(j * 16, 16)] = jnp.zeros((16,), jnp.float32)

    @pl.loop(0, RPT // K)
    def _(j):
        pltpu.sync_copy(rows0, acc_sh.at[pl.ds(s * RPT + j * K, K)])

    plsc.subcore_barrier()

    for j in range(4):
        idx_load(j, j)
    for b in range(2):
        idx_wait(b, b)
        pltpu.async_copy(hs_hbm.at[sxs[b]], rbs[b], gss[b])

    @pl.loop(0, T, step=4)
    def _(i0):
        for u in range(4):
            b = u % 2
            nu = (u + 2) % 4
            i = i0 + u
            pltpu.make_async_copy(hs_hbm.at[sxs[u]], rbs[b], gss[b]).wait()
            pltpu.sync_copy(rbs[b], acc_sh.at[dxs[u]], add=True)

            @pl.when(i + 4 < T)
            def _():
                idx_load(i + 4, u)

            @pl.when(i + 2 < T)
            def _():
                idx_wait(i + 2, nu)
                pltpu.async_copy(hs_hbm.at[sxs[nu]], rbs[b], gss[b])

    plsc.subcore_barrier()
    pltpu.sync_copy(acc_sh.at[pl.ds(s * RPT, RPT)],
                    out_hbm.at[c].at[pl.ds(s * RPT, RPT)])


BLK = 1000
GRID = N // BLK


def _dinv_of(da_ref, db_ref):
    deg = da_ref[:, :1] + db_ref[:, :1] + 1.0
    return lax.rsqrt(deg)


def _tc1_body(x_ref, w1_ref, da_ref, db_ref, hs_ref):
    dinv = _dinv_of(da_ref, db_ref)
    h = jnp.dot(x_ref[...], w1_ref[...], preferred_element_type=jnp.float32)
    hs_ref[...] = h * dinv


def _tc2_body(aa_ref, ab_ref, hs1_ref, da_ref, db_ref, b1_ref, w2_ref,
              hs2_ref):
    dinv = _dinv_of(da_ref, db_ref)
    conv1 = dinv * (aa_ref[...] + ab_ref[...] + hs1_ref[...]) + b1_ref[...]
    t = jnp.maximum(conv1, 0.0)
    h2 = jnp.dot(t, w2_ref[...], preferred_element_type=jnp.float32)
    hs2_ref[...] = h2 * dinv


def _tc3_body(aa_ref, ab_ref, hs2_ref, da_ref, db_ref, b2_ref, w3_ref,
              b3_ref, batch_ref, out_ref, cnt_ref):
    i = pl.program_id(0)
    dinv = _dinv_of(da_ref, db_ref)
    conv2 = dinv * (aa_ref[...] + ab_ref[...] + hs2_ref[...]) + b2_ref[...]
    t = jnp.maximum(conv2, 0.0)
    h3 = jnp.dot(t, w3_ref[...], preferred_element_type=jnp.float32)
    h3 = h3 + b3_ref[...]

    bb = batch_ref[0, 0, :]
    gids = lax.broadcasted_iota(jnp.int32, (BLK, G), 1)
    p = (bb[:, None] == gids).astype(jnp.float32)
    dn = (((0,), (0,)), ((), ()))
    partial = lax.dot_general(p, h3, dn, preferred_element_type=jnp.float32)
    ones_col = jnp.ones((BLK, 1), jnp.float32)
    cnt = lax.dot_general(p, ones_col, dn, preferred_element_type=jnp.float32)

    @pl.when(i == 0)
    def _():
        out_ref[...] = partial
        cnt_ref[...] = cnt

    @pl.when(i > 0)
    def _():
        out_ref[...] += partial
        cnt_ref[...] += cnt

    @pl.when(i == GRID - 1)
    def _():
        out_ref[...] = out_ref[...] / jnp.maximum(cnt_ref[...], 1.0)


def _row_spec(width):
    return pl.BlockSpec((BLK, width), lambda i: (i, 0))


def _full_spec(shape):
    nd = len(shape)
    return pl.BlockSpec(shape, lambda i: (0,) * nd)


_tc1 = pl.pallas_call(
    _tc1_body,
    grid=(GRID,),
    in_specs=[_row_spec(D), _full_spec((D, D)), _row_spec(16), _row_spec(16)],
    out_specs=_row_spec(D),
    out_shape=jax.ShapeDtypeStruct((N, D), jnp.float32),
)

_tc2 = pl.pallas_call(
    _tc2_body,
    grid=(GRID,),
    in_specs=[_row_spec(D), _row_spec(D), _row_spec(D), _row_spec(16),
              _row_spec(16), _full_spec((1, D)), _full_spec((D, D))],
    out_specs=_row_spec(D),
    out_shape=jax.ShapeDtypeStruct((N, D), jnp.float32),
)

_tc3 = pl.pallas_call(
    _tc3_body,
    grid=(GRID,),
    in_specs=[_row_spec(D), _row_spec(D), _row_spec(D), _row_spec(16),
              _row_spec(16), _full_spec((1, D)), _full_spec((D, DO)),
              _full_spec((1, DO)),
              pl.BlockSpec((1, 1, BLK), lambda i: (i, 0, 0))],
    out_specs=_full_spec((G, DO)),
    out_shape=jax.ShapeDtypeStruct((G, DO), jnp.float32),
    scratch_shapes=[pltpu.VMEM((G, 1), jnp.float32)],
)


def kernel(x, edge_index, batch, W1, b1, W2, b2, W3, b3):
    pad = EP - E
    src = jnp.concatenate([edge_index[0], jnp.zeros((pad,), jnp.int32)])
    dst = jnp.concatenate(
        [edge_index[1],
         N + (jnp.arange(pad, dtype=jnp.int32) % (NP - N))])
    b1r = b1.reshape(1, D)
    b2r = b2.reshape(1, D)
    b3r = b3.reshape(1, DO)
    batch3 = batch.reshape(GRID, 1, BLK)

    degp = _sc_degree(dst)
    da = degp[0, :N]
    db = degp[1, :N]

    hs1 = _tc1(x, W1, da, db)
    agg1 = _sc_aggregate(hs1, src, dst)
    hs2 = _tc2(agg1[0, :N], agg1[1, :N], hs1, da, db, b1r, W2)
    agg2 = _sc_aggregate(hs2, src, dst)
    out = _tc3(agg2[0, :N], agg2[1, :N], hs2, da, db, b2r, W3, b3r, batch3)
    return out


# speedup vs baseline: 2.8753x; 2.8753x over previous
import functools

import jax
import jax.numpy as jnp
from jax import lax
from jax.experimental import pallas as pl
from jax.experimental.pallas import tpu as pltpu
from jax.experimental.pallas import tpu_sc as plsc

N = 10000
E = 320000
G = 64
D = 128
DO = 64

NC = 2
NS = 16
K = 128
T = 80
EP = NC * NS * T * K
ECH = EP // K
NP = 10240
RPT = NP // NS

_mesh = plsc.VectorSubcoreMesh(core_axis_name="c", subcore_axis_name="s")


@functools.partial(
    pl.kernel,
    mesh=_mesh,
    out_type=jax.ShapeDtypeStruct((NC, NP, 16), jnp.float32),
    scratch_types=[
        pltpu.VMEM_SHARED((NP, 16), jnp.float32),
        pltpu.VMEM((K, 16), jnp.float32),
        pltpu.VMEM((K,), jnp.int32),
    ],
)
def _sc_degree(dst_hbm, out_hbm, acc_sh, ones_v, didx_v):
    c = lax.axis_index("c")
    s = lax.axis_index("s")
    base = (c * NS + s) * T * K

    @pl.loop(0, K)
    def _(r):
        ones_v[r, :] = jnp.zeros((16,), jnp.float32)

    @pl.loop(0, RPT // K)
    def _(j):
        pltpu.sync_copy(ones_v, acc_sh.at[pl.ds(s * RPT + j * K, K)])

    @pl.loop(0, K)
    def _(r):
        ones_v[r, :] = jnp.ones((16,), jnp.float32)

    plsc.subcore_barrier()

    @pl.loop(0, T)
    def _(i):
        pltpu.sync_copy(dst_hbm.at[pl.ds(base + i * K, K)], didx_v)
        pltpu.sync_copy(ones_v, acc_sh.at[didx_v], add=True)

    plsc.subcore_barrier()
    pltpu.sync_copy(acc_sh.at[pl.ds(s * RPT, RPT)],
                    out_hbm.at[c].at[pl.ds(s * RPT, RPT)])


@functools.partial(
    pl.kernel,
    mesh=_mesh,
    out_type=jax.ShapeDtypeStruct((NC, NP, D), jnp.float32),
    scratch_types=[
        pltpu.VMEM_SHARED((NP, D), jnp.float32),
        pltpu.VMEM((K, D), jnp.float32),
        pltpu.VMEM((K, D), jnp.float32),
        pltpu.VMEM((K,), jnp.int32),
        pltpu.VMEM((K,), jnp.int32),
        pltpu.VMEM((K,), jnp.int32),
        pltpu.VMEM((K,), jnp.int32),
        pltpu.VMEM((K,), jnp.int32),
        pltpu.VMEM((K,), jnp.int32),
        pltpu.VMEM((K,), jnp.int32),
        pltpu.VMEM((K,), jnp.int32),
        pltpu.SemaphoreType.DMA,
        pltpu.SemaphoreType.DMA,
        pltpu.SemaphoreType.DMA,
        pltpu.SemaphoreType.DMA,
        pltpu.SemaphoreType.DMA,
        pltpu.SemaphoreType.DMA,
    ],
)
def _sc_aggregate(hs_hbm, src_hbm, dst_hbm, out_hbm,
                  acc_sh, rows0, rows1,
                  sx0, sx1, sx2, sx3, dx0, dx1, dx2, dx3,
                  gs0, gs1, is0, is1, is2, is3):
    c = lax.axis_index("c")
    s = lax.axis_index("s")
    base = (c * NS + s) * T * K

    sxs = (sx0, sx1, sx2, sx3)
    dxs = (dx0, dx1, dx2, dx3)
    iss = (is0, is1, is2, is3)
    rbs = (rows0, rows1)
    gss = (gs0, gs1)

    def idx_load(j, slot):
        pltpu.async_copy(src_hbm.at[pl.ds(base + j * K, K)], sxs[slot],
                         iss[slot])
        pltpu.async_copy(dst_hbm.at[pl.ds(base + j * K, K)], dxs[slot],
                         iss[slot])

    def idx_wait(j, slot):
        pltpu.make_async_copy(src_hbm.at[pl.ds(base + j * K, K)], sxs[slot],
                              iss[slot]).wait()
        pltpu.make_async_copy(dst_hbm.at[pl.ds(base + j * K, K)], dxs[slot],
                              iss[slot]).wait()

    @pl.loop(0, K)
    def _(r):
        @pl.loop(0, D // 16)
        def _(j):
            rows0[r, pl.ds(j * 16, 16)] = jnp.zeros((16,), jnp.float32)

    @pl.loop(0, RPT // K)
    def _(j):
        pltpu.sync_copy(rows0, acc_sh.at[pl.ds(s * RPT + j * K, K)])

    plsc.subcore_barrier()

    for j in range(4):
        idx_load(j, j)
    for b in range(2):
        idx_wait(b, b)
        pltpu.async_copy(hs_hbm.at[sxs[b]], rbs[b], gss[b])

    @pl.loop(0, T, step=4)
    def _(i0):
        for u in range(4):
            b = u % 2
            nu = (u + 2) % 4
            i = i0 + u
            pltpu.make_async_copy(hs_hbm.at[sxs[u]], rbs[b], gss[b]).wait()
            pltpu.sync_copy(rbs[b], acc_sh.at[dxs[u]], add=True)

            @pl.when(i + 4 < T)
            def _():
                idx_load(i + 4, u)

            @pl.when(i + 2 < T)
            def _():
                idx_wait(i + 2, nu)
                pltpu.async_copy(hs_hbm.at[sxs[nu]], rbs[b], gss[b])

    plsc.subcore_barrier()
    pltpu.sync_copy(acc_sh.at[pl.ds(s * RPT, RPT)],
                    out_hbm.at[c].at[pl.ds(s * RPT, RPT)])


BLK = 1000
GRID = N // BLK


def _dinv_of(da_ref, db_ref):
    deg = da_ref[:, :1] + db_ref[:, :1] + 1.0
    return lax.rsqrt(deg)


def _tc1_body(x_ref, w1_ref, da_ref, db_ref, hs_ref):
    dinv = _dinv_of(da_ref, db_ref)
    h = jnp.dot(x_ref[...], w1_ref[...], preferred_element_type=jnp.float32)
    hs_ref[...] = h * dinv


def _tc2_body(aa_ref, ab_ref, hs1_ref, da_ref, db_ref, b1_ref, w2_ref,
              hs2_ref):
    dinv = _dinv_of(da_ref, db_ref)
    conv1 = dinv * (aa_ref[...] + ab_ref[...] + hs1_ref[...]) + b1_ref[...]
    t = jnp.maximum(conv1, 0.0)
    h2 = jnp.dot(t, w2_ref[...], preferred_element_type=jnp.float32)
    hs2_ref[...] = h2 * dinv


def _tc3_body(aa_ref, ab_ref, hs2_ref, da_ref, db_ref, b2_ref, w3_ref,
              b3_ref, batch_ref, out_ref, cnt_ref):
    i = pl.program_id(0)
    dinv = _dinv_of(da_ref, db_ref)
    conv2 = dinv * (aa_ref[...] + ab_ref[...] + hs2_ref[...]) + b2_ref[...]
    t = jnp.maximum(conv2, 0.0)
    h3 = jnp.dot(t, w3_ref[...], preferred_element_type=jnp.float32)
    h3 = h3 + b3_ref[...]

    bb = batch_ref[0, 0, :]
    gids = lax.broadcasted_iota(jnp.int32, (BLK, G), 1)
    p = (bb[:, None] == gids).astype(jnp.float32)
    dn = (((0,), (0,)), ((), ()))
    partial = lax.dot_general(p, h3, dn, preferred_element_type=jnp.float32)
    ones_col = jnp.ones((BLK, 1), jnp.float32)
    cnt = lax.dot_general(p, ones_col, dn, preferred_element_type=jnp.float32)

    @pl.when(i == 0)
    def _():
        out_ref[...] = partial
        cnt_ref[...] = cnt

    @pl.when(i > 0)
    def _():
        out_ref[...] += partial
        cnt_ref[...] += cnt

    @pl.when(i == GRID - 1)
    def _():
        out_ref[...] = out_ref[...] / jnp.maximum(cnt_ref[...], 1.0)


def _row_spec(width):
    return pl.BlockSpec((BLK, width), lambda i: (i, 0))


def _full_spec(shape):
    nd = len(shape)
    return pl.BlockSpec(shape, lambda i: (0,) * nd)


_tc1 = pl.pallas_call(
    _tc1_body,
    grid=(GRID,),
    in_specs=[_row_spec(D), _full_spec((D, D)), _row_spec(16), _row_spec(16)],
    out_specs=_row_spec(D),
    out_shape=jax.ShapeDtypeStruct((N, D), jnp.float32),
)

_tc2 = pl.pallas_call(
    _tc2_body,
    grid=(GRID,),
    in_specs=[_row_spec(D), _row_spec(D), _row_spec(D), _row_spec(16),
              _row_spec(16), _full_spec((1, D)), _full_spec((D, D))],
    out_specs=_row_spec(D),
    out_shape=jax.ShapeDtypeStruct((N, D), jnp.float32),
)

_tc3 = pl.pallas_call(
    _tc3_body,
    grid=(GRID,),
    in_specs=[_row_spec(D), _row_spec(D), _row_spec(D), _row_spec(16),
              _row_spec(16), _full_spec((1, D)), _full_spec((D, DO)),
              _full_spec((1, DO)),
              pl.BlockSpec((1, 1, BLK), lambda i: (i, 0, 0))],
    out_specs=_full_spec((G, DO)),
    out_shape=jax.ShapeDtypeStruct((G, DO), jnp.float32),
    scratch_shapes=[pltpu.VMEM((G, 1), jnp.float32)],
)


def kernel(x, edge_index, batch, W1, b1, W2, b2, W3, b3):
    pad = EP - E
    src = jnp.concatenate(
        [edge_index[0], jnp.arange(pad, dtype=jnp.int32) % N])
    dst = jnp.concatenate(
        [edge_index[1],
         N + (jnp.arange(pad, dtype=jnp.int32) % (NP - N))])
    b1r = b1.reshape(1, D)
    b2r = b2.reshape(1, D)
    b3r = b3.reshape(1, DO)
    batch3 = batch.reshape(GRID, 1, BLK)

    degp = _sc_degree(dst)
    da = degp[0, :N]
    db = degp[1, :N]

    hs1 = _tc1(x, W1, da, db)
    agg1 = _sc_aggregate(hs1, src, dst)
    hs2 = _tc2(agg1[0, :N], agg1[1, :N], hs1, da, db, b1r, W2)
    agg2 = _sc_aggregate(hs2, src, dst)
    out = _tc3(agg2[0, :N], agg2[1, :N], hs2, da, db, b2r, W3, b3r, batch3)
    return out


# speedup vs baseline: 3.5690x; 1.2413x over previous
import functools

import jax
import jax.numpy as jnp
from jax import lax
from jax.experimental import pallas as pl
from jax.experimental.pallas import tpu as pltpu
from jax.experimental.pallas import tpu_sc as plsc

N = 10000
E = 320000
G = 64
D = 128
DO = 64

NC = 2
NS = 16
K = 112
T = 90
EP = NC * NS * T * K
ECH = EP // K
NP = 10240
RPT = NP // NS

_mesh = plsc.VectorSubcoreMesh(core_axis_name="c", subcore_axis_name="s")


@functools.partial(
    pl.kernel,
    mesh=_mesh,
    out_type=jax.ShapeDtypeStruct((NC, NP, 16), jnp.float32),
    scratch_types=[
        pltpu.VMEM_SHARED((NP, 16), jnp.float32),
        pltpu.VMEM((K, 16), jnp.float32),
        pltpu.VMEM((K,), jnp.int32),
        pltpu.VMEM((K,), jnp.int32),
        pltpu.VMEM((K,), jnp.int32),
        pltpu.VMEM((K,), jnp.int32),
        pltpu.VMEM((K,), jnp.int32),
        pltpu.VMEM((K,), jnp.int32),
        pltpu.SemaphoreType.DMA,
        pltpu.SemaphoreType.DMA,
        pltpu.SemaphoreType.DMA,
        pltpu.SemaphoreType.DMA,
        pltpu.SemaphoreType.DMA,
        pltpu.SemaphoreType.DMA,
    ],
)
def _sc_degree(dst_hbm, out_hbm, acc_sh, ones_v,
               dx0, dx1, dx2, dx3, dx4, dx5,
               is0, is1, is2, is3, is4, is5):
    c = lax.axis_index("c")
    s = lax.axis_index("s")
    base = (c * NS + s) * T * K
    dxs = (dx0, dx1, dx2, dx3, dx4, dx5)
    iss = (is0, is1, is2, is3, is4, is5)

    def idx_load(j, slot):
        pltpu.async_copy(dst_hbm.at[pl.ds(base + j * K, K)], dxs[slot],
                         iss[slot])

    def idx_wait(j, slot):
        pltpu.make_async_copy(dst_hbm.at[pl.ds(base + j * K, K)], dxs[slot],
                              iss[slot]).wait()

    @pl.loop(0, K)
    def _(r):
        ones_v[r, :] = jnp.zeros((16,), jnp.float32)

    @pl.loop(0, 5)
    def _(j):
        pltpu.sync_copy(ones_v, acc_sh.at[pl.ds(s * RPT + j * K, K)])

    pltpu.sync_copy(ones_v.at[pl.ds(0, 80)],
                    acc_sh.at[pl.ds(s * RPT + 5 * K, 80)])

    @pl.loop(0, K)
    def _(r):
        ones_v[r, :] = jnp.ones((16,), jnp.float32)

    plsc.subcore_barrier()

    for j in range(6):
        idx_load(j, j)

    @pl.loop(0, T, step=6)
    def _(i0):
        for u in range(6):
            i = i0 + u
            idx_wait(i, u)
            pltpu.sync_copy(ones_v, acc_sh.at[dxs[u]], add=True)

            @pl.when(i + 6 < T)
            def _():
                idx_load(i + 6, u)

    plsc.subcore_barrier()
    pltpu.sync_copy(acc_sh.at[pl.ds(s * RPT, RPT)],
                    out_hbm.at[c].at[pl.ds(s * RPT, RPT)])


@functools.partial(
    pl.kernel,
    mesh=_mesh,
    out_type=jax.ShapeDtypeStruct((NC, NP, D), jnp.float32),
    scratch_types=(
        [pltpu.VMEM_SHARED((NP, D), jnp.float32)]
        + [pltpu.VMEM((K, D), jnp.float32)] * 3
        + [pltpu.VMEM((K,), jnp.int32)] * 12
        + [pltpu.SemaphoreType.DMA] * 12
    ),
)
def _sc_aggregate(hs_hbm, src_hbm, dst_hbm, out_hbm,
                  acc_sh, rows0, rows1, rows2,
                  sx0, sx1, sx2, sx3, sx4, sx5,
                  dx0, dx1, dx2, dx3, dx4, dx5,
                  gs0, gs1, gs2, ss0, ss1, ss2,
                  is0, is1, is2, is3, is4, is5):
    c = lax.axis_index("c")
    s = lax.axis_index("s")
    base = (c * NS + s) * T * K

    sxs = (sx0, sx1, sx2, sx3, sx4, sx5)
    dxs = (dx0, dx1, dx2, dx3, dx4, dx5)
    iss = (is0, is1, is2, is3, is4, is5)
    rbs = (rows0, rows1, rows2)
    gss = (gs0, gs1, gs2)
    sss = (ss0, ss1, ss2)

    def idx_load(j, slot):
        pltpu.async_copy(src_hbm.at[pl.ds(base + j * K, K)], sxs[slot],
                         iss[slot])
        pltpu.async_copy(dst_hbm.at[pl.ds(base + j * K, K)], dxs[slot],
                         iss[slot])

    def idx_wait(j, slot):
        pltpu.make_async_copy(src_hbm.at[pl.ds(base + j * K, K)], sxs[slot],
                              iss[slot]).wait()
        pltpu.make_async_copy(dst_hbm.at[pl.ds(base + j * K, K)], dxs[slot],
                              iss[slot]).wait()

    @pl.loop(0, K)
    def _(r):
        @pl.loop(0, D // 16)
        def _(j):
            rows0[r, pl.ds(j * 16, 16)] = jnp.zeros((16,), jnp.float32)

    @pl.loop(0, 5)
    def _(j):
        pltpu.sync_copy(rows0, acc_sh.at[pl.ds(s * RPT + j * K, K)])

    pltpu.sync_copy(rows0.at[pl.ds(0, 80)],
                    acc_sh.at[pl.ds(s * RPT + 5 * K, 80)])

    plsc.subcore_barrier()

    for j in range(6):
        idx_load(j, j)
    for b in range(2):
        idx_wait(b, b)
        pltpu.async_copy(hs_hbm.at[sxs[b]], rbs[b], gss[b])

    @pl.loop(0, T, step=6)
    def _(i0):
        for u in range(6):
            b = u % 3
            pb = (u + 2) % 3
            pu = (u + 5) % 6
            nu = (u + 2) % 6
            i = i0 + u
            pltpu.make_async_copy(hs_hbm.at[sxs[u]], rbs[b], gss[b]).wait()
            pltpu.async_copy(rbs[b], acc_sh.at[dxs[u]], sss[b], add=True)

            @pl.when(i >= 1)
            def _():
                pltpu.make_async_copy(rbs[pb], acc_sh.at[dxs[pu]],
                                      sss[pb]).wait()

            @pl.when((i >= 1) & (i + 5 < T))
            def _():
                idx_load(i + 5, pu)

            @pl.when(i + 2 < T)
            def _():
                idx_wait(i + 2, nu)
                pltpu.async_copy(hs_hbm.at[sxs[nu]], rbs[pb], gss[pb])

    pltpu.make_async_copy(rbs[(T - 1) % 3], acc_sh.at[dxs[(T - 1) % 6]],
                          sss[(T - 1) % 3]).wait()

    plsc.subcore_barrier()
    pltpu.sync_copy(acc_sh.at[pl.ds(s * RPT, RPT)],
                    out_hbm.at[c].at[pl.ds(s * RPT, RPT)])


BLK = 1000
GRID = N // BLK


def _dinv_of(dg_ref):
    deg = dg_ref[0, :, :1] + dg_ref[1, :, :1] + 1.0
    return lax.rsqrt(deg)


def _tc1_body(x_ref, w1_ref, dg_ref, hs_ref):
    dinv = _dinv_of(dg_ref)
    h = jnp.dot(x_ref[...], w1_ref[...], preferred_element_type=jnp.float32)
    hs_ref[...] = h * dinv


def _tc2_body(ag_ref, hs1_ref, dg_ref, b1_ref, w2_ref, hs2_ref):
    dinv = _dinv_of(dg_ref)
    conv1 = dinv * (ag_ref[0] + ag_ref[1] + hs1_ref[...]) + b1_ref[...]
    t = jnp.maximum(conv1, 0.0)
    h2 = jnp.dot(t, w2_ref[...], preferred_element_type=jnp.float32)
    hs2_ref[...] = h2 * dinv


def _tc3_body(ag_ref, hs2_ref, dg_ref, b2_ref, w3_ref,
              b3_ref, batch_ref, out_ref, cnt_ref):
    i = pl.program_id(0)
    dinv = _dinv_of(dg_ref)
    conv2 = dinv * (ag_ref[0] + ag_ref[1] + hs2_ref[...]) + b2_ref[...]
    t = jnp.maximum(conv2, 0.0)
    h3 = jnp.dot(t, w3_ref[...], preferred_element_type=jnp.float32)
    h3 = h3 + b3_ref[...]

    bb = batch_ref[0, 0, :]
    gids = lax.broadcasted_iota(jnp.int32, (BLK, G), 1)
    p = (bb[:, None] == gids).astype(jnp.float32)
    dn = (((0,), (0,)), ((), ()))
    partial = lax.dot_general(p, h3, dn, preferred_element_type=jnp.float32)
    ones_col = jnp.ones((BLK, 1), jnp.float32)
    cnt = lax.dot_general(p, ones_col, dn, preferred_element_type=jnp.float32)

    @pl.when(i == 0)
    def _():
        out_ref[...] = partial
        cnt_ref[...] = cnt

    @pl.when(i > 0)
    def _():
        out_ref[...] += partial
        cnt_ref[...] += cnt

    @pl.when(i == GRID - 1)
    def _():
        out_ref[...] = out_ref[...] / jnp.maximum(cnt_ref[...], 1.0)


def _row_spec(width):
    return pl.BlockSpec((BLK, width), lambda i: (i, 0))


def _pair_spec(width):
    return pl.BlockSpec((NC, BLK, width), lambda i: (0, i, 0))


def _full_spec(shape):
    nd = len(shape)
    return pl.BlockSpec(shape, lambda i: (0,) * nd)


_tc1 = pl.pallas_call(
    _tc1_body,
    grid=(GRID,),
    in_specs=[_row_spec(D), _full_spec((D, D)), _pair_spec(16)],
    out_specs=_row_spec(D),
    out_shape=jax.ShapeDtypeStruct((N, D), jnp.float32),
)

_tc2 = pl.pallas_call(
    _tc2_body,
    grid=(GRID,),
    in_specs=[_pair_spec(D), _row_spec(D), _pair_spec(16),
              _full_spec((1, D)), _full_spec((D, D))],
    out_specs=_row_spec(D),
    out_shape=jax.ShapeDtypeStruct((N, D), jnp.float32),
)

_tc3 = pl.pallas_call(
    _tc3_body,
    grid=(GRID,),
    in_specs=[_pair_spec(D), _row_spec(D), _pair_spec(16),
              _full_spec((1, D)), _full_spec((D, DO)),
              _full_spec((1, DO)),
              pl.BlockSpec((1, 1, BLK), lambda i: (i, 0, 0))],
    out_specs=_full_spec((G, DO)),
    out_shape=jax.ShapeDtypeStruct((G, DO), jnp.float32),
    scratch_shapes=[pltpu.VMEM((G, 1), jnp.float32)],
)


def kernel(x, edge_index, batch, W1, b1, W2, b2, W3, b3):
    pad = EP - E
    src = jnp.concatenate(
        [edge_index[0], jnp.arange(pad, dtype=jnp.int32) % N])
    dst = jnp.concatenate(
        [edge_index[1],
         N + (jnp.arange(pad, dtype=jnp.int32) % (NP - N))])
    b1r = b1.reshape(1, D)
    b2r = b2.reshape(1, D)
    b3r = b3.reshape(1, DO)
    batch3 = batch.reshape(GRID, 1, BLK)

    dg = _sc_degree(dst)

    hs1 = _tc1(x, W1, dg)
    ag1 = _sc_aggregate(hs1, src, dst)
    hs2 = _tc2(ag1, hs1, dg, b1r, W2)
    ag2 = _sc_aggregate(hs2, src, dst)
    out = _tc3(ag2, hs2, dg, b2r, W3, b3r, batch3)
    return out
